# baseline (speedup 1.0000x reference)
import jax
import jax.numpy as jnp
from jax import lax
from jax.experimental import pallas as pl
from jax.experimental.pallas import tpu as pltpu
from jax.experimental.pallas import tpu_sc as plsc

NN = 10000
NPAD = 10240
EE = 160000
D = 128
BLK = 128
NBLK = EE // BLK
NC = 2
NS = 16
NW = NC * NS
L = 16
PER_TILE = NPAD // NS
KMAX = 40
CSTG = 10
NSTG = NBLK // CSTG
DSTG = 10
NDROW = 2 * EE // BLK


def _tc_prep_body(node_ref, basis_ref, att_ref, mf_ref, hp_ref, hn_ref,
                  scal_ref):
    x = node_ref[...]
    att = att_ref[...]
    b0 = basis_ref[0]
    b1 = basis_ref[1]
    w0 = att[0:1, 0:1] * b0 + att[0:1, 1:2] * b1
    w1 = att[1:2, 0:1] * b0 + att[1:2, 1:2] * b1
    hp = jnp.dot(x, w0, preferred_element_type=jnp.float32)
    hn = jnp.dot(x, w1, preferred_element_type=jnp.float32)
    hp_ref[...] = hp
    hn_ref[...] = hn
    mf = mf_ref[...]
    mfa = mf[:, :D]
    mfb = mf[:, D:]
    ap = jnp.sum(hp * mfa, axis=1)
    bp = jnp.sum(hp * mfb, axis=1)
    an = jnp.sum(hn * mfa, axis=1)
    bn = jnp.sum(hn * mfb, axis=1)
    scal_ref[...] = jnp.concatenate(
        [ap[None], bp[None], an[None], bn[None]], axis=0)[None]


def _tc_prep(node_reps, basis, att, mapping_func):
    blk = 1000
    grid = NN // blk
    return pl.pallas_call(
        _tc_prep_body,
        grid=(grid,),
        in_specs=[
            pl.BlockSpec((blk, D), lambda i: (i, 0)),
            pl.BlockSpec((2, D, D), lambda i: (0, 0, 0)),
            pl.BlockSpec((2, 2), lambda i: (0, 0)),
            pl.BlockSpec((1, 2 * D), lambda i: (0, 0)),
        ],
        out_specs=[
            pl.BlockSpec((blk, D), lambda i: (i, 0)),
            pl.BlockSpec((blk, D), lambda i: (i, 0)),
            pl.BlockSpec((1, 4, blk), lambda i: (i, 0, 0)),
        ],
        out_shape=[
            jax.ShapeDtypeStruct((NN, D), jnp.float32),
            jax.ShapeDtypeStruct((NN, D), jnp.float32),
            jax.ShapeDtypeStruct((NN // blk, 4, blk), jnp.float32),
        ],
    )(node_reps, basis, att, mapping_func)


def _sc_deg_body(srcs_hbm, out_hbm, idx_v, ones_v, zb_v, deg_sh, sem):
    c = lax.axis_index("c")
    s = lax.axis_index("s")
    w = c * NS + s
    for i in range(PER_TILE // L):
        zb_v[pl.ds(i * L, L)] = jnp.zeros((L,), jnp.float32)
    for i in range(BLK // L):
        ones_v[pl.ds(i * L, L)] = jnp.ones((L,), jnp.float32)
    pltpu.sync_copy(zb_v, deg_sh.at[pl.ds(s * PER_TILE, PER_TILE)])
    plsc.subcore_barrier()

    @pl.loop(w, NDROW // DSTG, step=NW)
    def _(t):
        pltpu.sync_copy(srcs_hbm.at[t], idx_v)
        for k in range(DSTG):
            pltpu.make_async_copy(
                ones_v, deg_sh.at[idx_v.at[k]], sem).start(add=True)
        for k in range(DSTG):
            pltpu.make_async_copy(
                ones_v, deg_sh.at[idx_v.at[k]], sem).wait()

    plsc.subcore_barrier()
    pltpu.sync_copy(deg_sh.at[pl.ds(s * PER_TILE, PER_TILE)],
                    out_hbm.at[c, pl.ds(s * PER_TILE, PER_TILE)])


def _sc_deg(srcs2d):
    mesh = plsc.VectorSubcoreMesh(core_axis_name="c", subcore_axis_name="s",
                                  num_cores=NC, num_subcores=NS)
    f = pl.kernel(
        _sc_deg_body,
        out_type=jax.ShapeDtypeStruct((NC, NPAD), jnp.float32),
        mesh=mesh,
        compiler_params=pltpu.CompilerParams(needs_layout_passes=False),
        scratch_types=[
            pltpu.VMEM((DSTG, BLK), jnp.int32),
            pltpu.VMEM((BLK,), jnp.float32),
            pltpu.VMEM((PER_TILE,), jnp.float32),
            pltpu.VMEM_SHARED((NPAD,), jnp.float32),
            pltpu.SemaphoreType.DMA,
        ],
    )
    return f(srcs2d)


def _sc_coef_body(adjp_hbm, adjn_hbm, scal_hbm, deg_hbm, pkp_hbm, pkn_hbm,
                  tab_a, tab_b, tab_inv, tab_d1, idx_v, pkv):
    c = lax.axis_index("c")
    s = lax.axis_index("s")
    w = c * NS + s
    pltpu.sync_copy(deg_hbm.at[0], tab_inv)
    pltpu.sync_copy(deg_hbm.at[1], tab_d1)

    @pl.loop(0, NPAD // L)
    def _(i):
        sl = pl.ds(i * L, L)
        x = tab_inv[sl] + tab_d1[sl]
        y = plsc.bitcast(
            jnp.int32(0x5F3759DF) - (plsc.bitcast(x, jnp.int32) >> 1),
            jnp.float32)
        hx = 0.5 * x
        y = y * (1.5 - hx * y * y)
        y = y * (1.5 - hx * y * y)
        y = y * (1.5 - hx * y * y)
        tab_inv[sl] = y

    def sign_pass(adj_hbm, pk_hbm, r0, r1, sign):
        pltpu.sync_copy(scal_hbm.at[r0], tab_a)
        pltpu.sync_copy(scal_hbm.at[r1], tab_b)

        @pl.loop(w, NSTG, step=NW)
        def _(t):
            base = t * (CSTG * BLK)
            pltpu.sync_copy(adj_hbm.at[:, pl.ds(base, CSTG * BLK)], idx_v)
            for k in range(CSTG):
                for i in range(BLK // L):
                    off = k * BLK + i * L
                    s16 = idx_v[0, pl.ds(off, L)]
                    d16 = idx_v[1, pl.ds(off, L)]
                    va = plsc.load_gather(tab_a, [s16])
                    vb = plsc.load_gather(tab_b, [d16])
                    x = va + vb
                    x = jnp.maximum(x, 0.2 * x)
                    sg = sign / (1.0 + jnp.exp(-x))
                    vi = plsc.load_gather(tab_inv, [d16])
                    sl = pl.ds(i * L, L)
                    pkv[0, sl] = s16
                    pkv[1, sl] = d16
                    pkv[2, sl] = plsc.bitcast(sg * vi, jnp.int32)
                pltpu.sync_copy(pkv, pk_hbm.at[t * CSTG + k])

    sign_pass(adjp_hbm, pkp_hbm, 0, 1, 1.0)
    sign_pass(adjn_hbm, pkn_hbm, 2, 3, -1.0)


def _sc_coef(adj_pos, adj_neg, scal, degp):
    mesh = plsc.VectorSubcoreMesh(core_axis_name="c", subcore_axis_name="s",
                                  num_cores=NC, num_subcores=NS)
    f = pl.kernel(
        _sc_coef_body,
        out_type=[
            jax.ShapeDtypeStruct((NBLK, 3, BLK), jnp.int32),
            jax.ShapeDtypeStruct((NBLK, 3, BLK), jnp.int32),
        ],
        mesh=mesh,
        compiler_params=pltpu.CompilerParams(needs_layout_passes=False),
        scratch_types=[
            pltpu.VMEM((NN,), jnp.float32),
            pltpu.VMEM((NN,), jnp.float32),
            pltpu.VMEM((NPAD,), jnp.float32),
            pltpu.VMEM((NPAD,), jnp.float32),
            pltpu.VMEM((2, CSTG * BLK), jnp.int32),
            pltpu.VMEM((3, BLK), jnp.int32),
        ],
    )
    return f(adj_pos, adj_neg, scal, degp)


def _sc_main_body(pkp_hbm, pkn_hbm, hp_hbm, hn_hbm, out_hbm,
                  pk0, pk1, pk2, pk3, rows0, rows1, acc_sh,
                  sp0, sp1, sp2, sp3, sg0, sg1, ss0, ss1):
    c = lax.axis_index("c")
    s = lax.axis_index("s")
    w = c * NS + s
    pkb = (pk0, pk1, pk2, pk3)
    spk = (sp0, sp1, sp2, sp3)
    rwb = (rows0, rows1)
    sgt = (sg0, sg1)
    ssb = (ss0, ss1)

    @pl.loop(0, BLK)
    def _(i):
        for j in range(D // L):
            rows0[i, pl.ds(j * L, L)] = jnp.zeros((L,), jnp.float32)

    for k in range(PER_TILE // BLK):
        pltpu.sync_copy(rows0, acc_sh.at[pl.ds(s * PER_TILE + k * BLK, BLK)])
    plsc.subcore_barrier()

    def phase(pk_hbm, h_hbm):
        def pk_cp(b, i):
            return pltpu.make_async_copy(pk_hbm.at[b], pkb[i], spk[i])

        def g_cp2(pi, ri):
            return pltpu.make_async_copy(
                h_hbm.at[pkb[pi].at[1]], rwb[ri], sgt[ri])

        def sc_cp(pi, ri):
            return pltpu.make_async_copy(
                rwb[ri], acc_sh.at[pkb[pi].at[0]], ssb[ri])

        def proc(pi, ri):
            g_cp2(pi, ri).wait()
            two = jnp.broadcast_to(jnp.int32(2), (L,))

            @pl.loop(0, BLK, unroll=4)
            def _(e):
                e16 = jnp.broadcast_to(e, (L,)).astype(jnp.int32)
                cv = plsc.bitcast(
                    plsc.load_gather(pkb[pi], [two, e16]), jnp.float32)
                for j in range(D // L):
                    sl = pl.ds(j * L, L)
                    rwb[ri][e, sl] = rwb[ri][e, sl] * cv

            sc_cp(pi, ri).start(add=True)

        for q in range(4):
            pk_cp(w + q * NW, q).start()
        pk_cp(w, 0).wait()
        g_cp2(0, 0).start()

        @pl.loop(0, KMAX // 4)
        def _(q):
            b0 = w + (4 * q) * NW
            for u in range(4):
                bu = b0 + u * NW
                bn = bu + NW
                bp = bu - NW

                @pl.when(bn < NBLK)
                def _(bn=bn, bp=bp, u=u):
                    pk_cp(bn, (u + 1) % 4).wait()

                    @pl.when(bp >= 0)
                    def _(bp=bp, u=u):
                        sc_cp((u - 1) % 4, (u + 1) % 2).wait()

                    g_cp2((u + 1) % 4, (u + 1) % 2).start()

                @pl.when(bu < NBLK)
                def _(bu=bu, u=u):
                    proc(u % 4, u % 2)

                @pl.when((bp >= 0) & (bp + 4 * NW < NBLK))
                def _(bp=bp, u=u):
                    pk_cp(bp + 4 * NW, (u - 1) % 4).start()

        for v in range(KMAX):
            bv = w + v * NW

            @pl.when((bv < NBLK) & (bv + 2 * NW >= NBLK))
            def _(v=v):
                sc_cp(v % 4, v % 2).wait()

    phase(pkp_hbm, hp_hbm)
    phase(pkn_hbm, hn_hbm)
    plsc.subcore_barrier()
    pltpu.sync_copy(acc_sh.at[pl.ds(s * PER_TILE, PER_TILE)],
                    out_hbm.at[c, pl.ds(s * PER_TILE, PER_TILE)])


def _sc_main(pk_pos, pk_neg, h_pos, h_neg):
    mesh = plsc.VectorSubcoreMesh(core_axis_name="c", subcore_axis_name="s",
                                  num_cores=NC, num_subcores=NS)
    f = pl.kernel(
        _sc_main_body,
        out_type=jax.ShapeDtypeStruct((NC, NPAD, D), jnp.float32),
        mesh=mesh,
        compiler_params=pltpu.CompilerParams(needs_layout_passes=False),
        scratch_types=[
            pltpu.VMEM((3, BLK), jnp.int32),
            pltpu.VMEM((3, BLK), jnp.int32),
            pltpu.VMEM((3, BLK), jnp.int32),
            pltpu.VMEM((3, BLK), jnp.int32),
            pltpu.VMEM((BLK, D), jnp.float32),
            pltpu.VMEM((BLK, D), jnp.float32),
            pltpu.VMEM_SHARED((NPAD, D), jnp.float32),
            pltpu.SemaphoreType.DMA,
            pltpu.SemaphoreType.DMA,
            pltpu.SemaphoreType.DMA,
            pltpu.SemaphoreType.DMA,
            pltpu.SemaphoreType.DMA,
            pltpu.SemaphoreType.DMA,
            pltpu.SemaphoreType.DMA,
            pltpu.SemaphoreType.DMA,
        ],
    )
    return f(pk_pos, pk_neg, h_pos, h_neg)


def _tc_final_body(p_ref, deg_ref, bias_ref, out_ref):
    deg = deg_ref[0] + deg_ref[1]
    inv = jnp.where(deg > 0.0, lax.rsqrt(deg), 0.0)
    out_ref[...] = (p_ref[0] + p_ref[1]) * inv[:, None] + bias_ref[...]


def _tc_final(partials, degp, bias):
    blk = 2048
    grid = NPAD // blk
    return pl.pallas_call(
        _tc_final_body,
        grid=(grid,),
        in_specs=[
            pl.BlockSpec((NC, blk, D), lambda i: (0, i, 0)),
            pl.BlockSpec((NC, blk), lambda i: (0, i)),
            pl.BlockSpec((1, D), lambda i: (0, 0)),
        ],
        out_specs=pl.BlockSpec((blk, D), lambda i: (i, 0)),
        out_shape=jax.ShapeDtypeStruct((NPAD, D), jnp.float32),
    )(partials, degp, bias)


def kernel(node_reps, adj_pos, adj_neg, basis, att, mapping_func, bias):
    h_pos, h_neg, scal3 = _tc_prep(node_reps, basis, att, mapping_func)
    scal = scal3.transpose(1, 0, 2).reshape(4, NN)
    srcs2d = jnp.concatenate([adj_pos[0], adj_neg[0]]).reshape(
        NDROW // DSTG, DSTG, BLK)
    degp = _sc_deg(srcs2d)
    pk_pos, pk_neg = _sc_coef(adj_pos, adj_neg, scal, degp)
    partials = _sc_main(pk_pos, pk_neg, h_pos, h_neg)
    return _tc_final(partials, degp, bias)[:NN]


# speedup vs baseline: 1.0352x; 1.0352x over previous
import jax
import jax.numpy as jnp
from jax import lax
from jax.experimental import pallas as pl
from jax.experimental.pallas import tpu as pltpu
from jax.experimental.pallas import tpu_sc as plsc

NN = 10000
NPAD = 10240
EE = 160000
D = 128
BLK = 128
NBLK = EE // BLK
NC = 2
NS = 16
NW = NC * NS
L = 16
PER_TILE = NPAD // NS
KMAX = 40
CSTG = 10
NSTG = NBLK // CSTG
DSTG = 10
NDROW = 2 * EE // BLK


def _tc_prep_body(node_ref, basis_ref, att_ref, mf_ref, hp_ref, hn_ref,
                  scal_ref):
    x = node_ref[...]
    att = att_ref[...]
    b0 = basis_ref[0]
    b1 = basis_ref[1]
    w0 = att[0:1, 0:1] * b0 + att[0:1, 1:2] * b1
    w1 = att[1:2, 0:1] * b0 + att[1:2, 1:2] * b1
    hp = jnp.dot(x, w0, preferred_element_type=jnp.float32)
    hn = jnp.dot(x, w1, preferred_element_type=jnp.float32)
    hp_ref[...] = hp
    hn_ref[...] = hn
    mf = mf_ref[...]
    mfa = mf[:, :D]
    mfb = mf[:, D:]
    ap = jnp.sum(hp * mfa, axis=1)
    bp = jnp.sum(hp * mfb, axis=1)
    an = jnp.sum(hn * mfa, axis=1)
    bn = jnp.sum(hn * mfb, axis=1)
    scal_ref[...] = jnp.concatenate(
        [ap[None], bp[None], an[None], bn[None]], axis=0)[None]


def _tc_prep(node_reps, basis, att, mapping_func):
    blk = 1000
    grid = NN // blk
    return pl.pallas_call(
        _tc_prep_body,
        grid=(grid,),
        in_specs=[
            pl.BlockSpec((blk, D), lambda i: (i, 0)),
            pl.BlockSpec((2, D, D), lambda i: (0, 0, 0)),
            pl.BlockSpec((2, 2), lambda i: (0, 0)),
            pl.BlockSpec((1, 2 * D), lambda i: (0, 0)),
        ],
        out_specs=[
            pl.BlockSpec((blk, D), lambda i: (i, 0)),
            pl.BlockSpec((blk, D), lambda i: (i, 0)),
            pl.BlockSpec((1, 4, blk), lambda i: (i, 0, 0)),
        ],
        out_shape=[
            jax.ShapeDtypeStruct((NN, D), jnp.float32),
            jax.ShapeDtypeStruct((NN, D), jnp.float32),
            jax.ShapeDtypeStruct((NN // blk, 4, blk), jnp.float32),
        ],
    )(node_reps, basis, att, mapping_func)


def _sc_deg_body(srcs_hbm, out_hbm, idx_v, ones_v, zb_v, deg_sh, sem):
    c = lax.axis_index("c")
    s = lax.axis_index("s")
    w = c * NS + s
    for i in range(PER_TILE // L):
        zb_v[pl.ds(i * L, L)] = jnp.zeros((L,), jnp.float32)
    for i in range(BLK // L):
        ones_v[pl.ds(i * L, L)] = jnp.ones((L,), jnp.float32)
    pltpu.sync_copy(zb_v, deg_sh.at[pl.ds(s * PER_TILE, PER_TILE)])
    plsc.subcore_barrier()

    @pl.loop(w, NDROW // DSTG, step=NW)
    def _(t):
        pltpu.sync_copy(srcs_hbm.at[t], idx_v)
        for k in range(DSTG):
            pltpu.make_async_copy(
                ones_v, deg_sh.at[idx_v.at[k]], sem).start(add=True)
        for k in range(DSTG):
            pltpu.make_async_copy(
                ones_v, deg_sh.at[idx_v.at[k]], sem).wait()

    plsc.subcore_barrier()
    pltpu.sync_copy(deg_sh.at[pl.ds(s * PER_TILE, PER_TILE)],
                    out_hbm.at[c, pl.ds(s * PER_TILE, PER_TILE)])


def _sc_deg(srcs2d):
    mesh = plsc.VectorSubcoreMesh(core_axis_name="c", subcore_axis_name="s",
                                  num_cores=NC, num_subcores=NS)
    f = pl.kernel(
        _sc_deg_body,
        out_type=jax.ShapeDtypeStruct((NC, NPAD), jnp.float32),
        mesh=mesh,
        compiler_params=pltpu.CompilerParams(needs_layout_passes=False),
        scratch_types=[
            pltpu.VMEM((DSTG, BLK), jnp.int32),
            pltpu.VMEM((BLK,), jnp.float32),
            pltpu.VMEM((PER_TILE,), jnp.float32),
            pltpu.VMEM_SHARED((NPAD,), jnp.float32),
            pltpu.SemaphoreType.DMA,
        ],
    )
    return f(srcs2d)


def _sc_coef_body(adjp_hbm, adjn_hbm, scal_hbm, deg_hbm, pkp_hbm, pkn_hbm,
                  tab_a, tab_b, tab_inv, d0_v, d1_v, inv_sh, idx_v, pkv):
    c = lax.axis_index("c")
    s = lax.axis_index("s")
    w = c * NS + s
    pltpu.sync_copy(deg_hbm.at[0, pl.ds(s * PER_TILE, PER_TILE)], d0_v)
    pltpu.sync_copy(deg_hbm.at[1, pl.ds(s * PER_TILE, PER_TILE)], d1_v)

    @pl.loop(0, PER_TILE // L)
    def _(i):
        sl = pl.ds(i * L, L)
        x = d0_v[sl] + d1_v[sl]
        y = plsc.bitcast(
            jnp.int32(0x5F3759DF) - (plsc.bitcast(x, jnp.int32) >> 1),
            jnp.float32)
        hx = 0.5 * x
        y = y * (1.5 - hx * y * y)
        y = y * (1.5 - hx * y * y)
        y = y * (1.5 - hx * y * y)
        d0_v[sl] = y

    pltpu.sync_copy(d0_v, inv_sh.at[pl.ds(s * PER_TILE, PER_TILE)])
    plsc.subcore_barrier()
    pltpu.sync_copy(inv_sh, tab_inv)

    def sign_pass(adj_hbm, pk_hbm, r0, r1, sign):
        pltpu.sync_copy(scal_hbm.at[r0], tab_a)
        pltpu.sync_copy(scal_hbm.at[r1], tab_b)

        @pl.loop(w, NSTG, step=NW)
        def _(t):
            base = t * (CSTG * BLK)
            pltpu.sync_copy(adj_hbm.at[:, pl.ds(base, CSTG * BLK)], idx_v)
            for k in range(CSTG):
                for i in range(BLK // L):
                    off = k * BLK + i * L
                    s16 = idx_v[0, pl.ds(off, L)]
                    d16 = idx_v[1, pl.ds(off, L)]
                    va = plsc.load_gather(tab_a, [s16])
                    vb = plsc.load_gather(tab_b, [d16])
                    x = va + vb
                    x = jnp.maximum(x, 0.2 * x)
                    sg = sign / (1.0 + jnp.exp(-x))
                    vi = plsc.load_gather(tab_inv, [d16])
                    sl = pl.ds(i * L, L)
                    pkv[0, sl] = s16
                    pkv[1, sl] = d16
                    pkv[2, sl] = plsc.bitcast(sg * vi, jnp.int32)
                pltpu.sync_copy(pkv, pk_hbm.at[t * CSTG + k])

    sign_pass(adjp_hbm, pkp_hbm, 0, 1, 1.0)
    sign_pass(adjn_hbm, pkn_hbm, 2, 3, -1.0)


def _sc_coef(adj_pos, adj_neg, scal, degp):
    mesh = plsc.VectorSubcoreMesh(core_axis_name="c", subcore_axis_name="s",
                                  num_cores=NC, num_subcores=NS)
    f = pl.kernel(
        _sc_coef_body,
        out_type=[
            jax.ShapeDtypeStruct((NBLK, 3, BLK), jnp.int32),
            jax.ShapeDtypeStruct((NBLK, 3, BLK), jnp.int32),
        ],
        mesh=mesh,
        compiler_params=pltpu.CompilerParams(needs_layout_passes=False),
        scratch_types=[
            pltpu.VMEM((NN,), jnp.float32),
            pltpu.VMEM((NN,), jnp.float32),
            pltpu.VMEM((NPAD,), jnp.float32),
            pltpu.VMEM((PER_TILE,), jnp.float32),
            pltpu.VMEM((PER_TILE,), jnp.float32),
            pltpu.VMEM_SHARED((NPAD,), jnp.float32),
            pltpu.VMEM((2, CSTG * BLK), jnp.int32),
            pltpu.VMEM((3, BLK), jnp.int32),
        ],
    )
    return f(adj_pos, adj_neg, scal, degp)


def _sc_main_body(pkp_hbm, pkn_hbm, hp_hbm, hn_hbm, out_hbm,
                  pk0, pk1, pk2, pk3, rows0, rows1, acc_sh,
                  sp0, sp1, sp2, sp3, sg0, sg1, ss0, ss1):
    c = lax.axis_index("c")
    s = lax.axis_index("s")
    w = c * NS + s
    pkb = (pk0, pk1, pk2, pk3)
    spk = (sp0, sp1, sp2, sp3)
    rwb = (rows0, rows1)
    sgt = (sg0, sg1)
    ssb = (ss0, ss1)

    @pl.loop(0, BLK)
    def _(i):
        for j in range(D // L):
            rows0[i, pl.ds(j * L, L)] = jnp.zeros((L,), jnp.float32)

    for k in range(PER_TILE // BLK):
        pltpu.sync_copy(rows0, acc_sh.at[pl.ds(s * PER_TILE + k * BLK, BLK)])
    plsc.subcore_barrier()

    def phase(pk_hbm, h_hbm):
        def pk_cp(b, i):
            return pltpu.make_async_copy(pk_hbm.at[b], pkb[i], spk[i])

        def g_cp2(pi, ri):
            return pltpu.make_async_copy(
                h_hbm.at[pkb[pi].at[1]], rwb[ri], sgt[ri])

        def sc_cp(pi, ri):
            return pltpu.make_async_copy(
                rwb[ri], acc_sh.at[pkb[pi].at[0]], ssb[ri])

        def proc(pi, ri):
            g_cp2(pi, ri).wait()
            two = jnp.broadcast_to(jnp.int32(2), (L,))

            @pl.loop(0, BLK, unroll=4)
            def _(e):
                e16 = jnp.broadcast_to(e, (L,)).astype(jnp.int32)
                cv = plsc.bitcast(
                    plsc.load_gather(pkb[pi], [two, e16]), jnp.float32)
                for j in range(D // L):
                    sl = pl.ds(j * L, L)
                    rwb[ri][e, sl] = rwb[ri][e, sl] * cv

            sc_cp(pi, ri).start(add=True)

        for q in range(4):
            pk_cp(w + q * NW, q).start()
        pk_cp(w, 0).wait()
        g_cp2(0, 0).start()

        @pl.loop(0, KMAX // 4)
        def _(q):
            b0 = w + (4 * q) * NW
            for u in range(4):
                bu = b0 + u * NW
                bn = bu + NW
                bp = bu - NW

                @pl.when(bn < NBLK)
                def _(bn=bn, bp=bp, u=u):
                    pk_cp(bn, (u + 1) % 4).wait()

                    @pl.when(bp >= 0)
                    def _(bp=bp, u=u):
                        sc_cp((u - 1) % 4, (u + 1) % 2).wait()

                    g_cp2((u + 1) % 4, (u + 1) % 2).start()

                @pl.when(bu < NBLK)
                def _(bu=bu, u=u):
                    proc(u % 4, u % 2)

                @pl.when((bp >= 0) & (bp + 4 * NW < NBLK))
                def _(bp=bp, u=u):
                    pk_cp(bp + 4 * NW, (u - 1) % 4).start()

        for v in range(KMAX):
            bv = w + v * NW

            @pl.when((bv < NBLK) & (bv + 2 * NW >= NBLK))
            def _(v=v):
                sc_cp(v % 4, v % 2).wait()

    phase(pkp_hbm, hp_hbm)
    phase(pkn_hbm, hn_hbm)
    plsc.subcore_barrier()
    pltpu.sync_copy(acc_sh.at[pl.ds(s * PER_TILE, PER_TILE)],
                    out_hbm.at[c, pl.ds(s * PER_TILE, PER_TILE)])


def _sc_main(pk_pos, pk_neg, h_pos, h_neg):
    mesh = plsc.VectorSubcoreMesh(core_axis_name="c", subcore_axis_name="s",
                                  num_cores=NC, num_subcores=NS)
    f = pl.kernel(
        _sc_main_body,
        out_type=jax.ShapeDtypeStruct((NC, NPAD, D), jnp.float32),
        mesh=mesh,
        compiler_params=pltpu.CompilerParams(needs_layout_passes=False),
        scratch_types=[
            pltpu.VMEM((3, BLK), jnp.int32),
            pltpu.VMEM((3, BLK), jnp.int32),
            pltpu.VMEM((3, BLK), jnp.int32),
            pltpu.VMEM((3, BLK), jnp.int32),
            pltpu.VMEM((BLK, D), jnp.float32),
            pltpu.VMEM((BLK, D), jnp.float32),
            pltpu.VMEM_SHARED((NPAD, D), jnp.float32),
            pltpu.SemaphoreType.DMA,
            pltpu.SemaphoreType.DMA,
            pltpu.SemaphoreType.DMA,
            pltpu.SemaphoreType.DMA,
            pltpu.SemaphoreType.DMA,
            pltpu.SemaphoreType.DMA,
            pltpu.SemaphoreType.DMA,
            pltpu.SemaphoreType.DMA,
        ],
    )
    return f(pk_pos, pk_neg, h_pos, h_neg)


def _tc_final_body(p_ref, deg_ref, bias_ref, out_ref):
    deg = deg_ref[0] + deg_ref[1]
    inv = jnp.where(deg > 0.0, lax.rsqrt(deg), 0.0)
    out_ref[...] = (p_ref[0] + p_ref[1]) * inv[:, None] + bias_ref[...]


def _tc_final(partials, degp, bias):
    blk = 2048
    grid = NPAD // blk
    return pl.pallas_call(
        _tc_final_body,
        grid=(grid,),
        in_specs=[
            pl.BlockSpec((NC, blk, D), lambda i: (0, i, 0)),
            pl.BlockSpec((NC, blk), lambda i: (0, i)),
            pl.BlockSpec((1, D), lambda i: (0, 0)),
        ],
        out_specs=pl.BlockSpec((blk, D), lambda i: (i, 0)),
        out_shape=jax.ShapeDtypeStruct((NPAD, D), jnp.float32),
    )(partials, degp, bias)


def kernel(node_reps, adj_pos, adj_neg, basis, att, mapping_func, bias):
    h_pos, h_neg, scal3 = _tc_prep(node_reps, basis, att, mapping_func)
    scal = scal3.transpose(1, 0, 2).reshape(4, NN)
    srcs2d = jnp.concatenate([adj_pos[0], adj_neg[0]]).reshape(
        NDROW // DSTG, DSTG, BLK)
    degp = _sc_deg(srcs2d)
    pk_pos, pk_neg = _sc_coef(adj_pos, adj_neg, scal, degp)
    partials = _sc_main(pk_pos, pk_neg, h_pos, h_neg)
    return _tc_final(partials, degp, bias)[:NN]


# speedup vs baseline: 1.0497x; 1.0140x over previous
import jax
import jax.numpy as jnp
from jax import lax
from jax.experimental import pallas as pl
from jax.experimental.pallas import tpu as pltpu
from jax.experimental.pallas import tpu_sc as plsc

NN = 10000
NPAD = 10240
EE = 160000
D = 128
BLK = 128
NBLK = EE // BLK
NC = 2
NS = 16
NW = NC * NS
L = 16
PER_TILE = NPAD // NS
KMAX = 40
CSTG = 10
NSTG = NBLK // CSTG
DSTG = 10
NDROW = 2 * EE // BLK


def _tc_prep_body(node_ref, basis_ref, att_ref, mf_ref, hp_ref, hn_ref,
                  scal_ref):
    x = node_ref[...]
    att = att_ref[...]
    b0 = basis_ref[0]
    b1 = basis_ref[1]
    w0 = att[0:1, 0:1] * b0 + att[0:1, 1:2] * b1
    w1 = att[1:2, 0:1] * b0 + att[1:2, 1:2] * b1
    hp = jnp.dot(x, w0, preferred_element_type=jnp.float32)
    hn = jnp.dot(x, w1, preferred_element_type=jnp.float32)
    hp_ref[...] = hp
    hn_ref[...] = hn
    mf = mf_ref[...]
    mfa = mf[:, :D]
    mfb = mf[:, D:]
    ap = jnp.sum(hp * mfa, axis=1)
    bp = jnp.sum(hp * mfb, axis=1)
    an = jnp.sum(hn * mfa, axis=1)
    bn = jnp.sum(hn * mfb, axis=1)
    scal_ref[...] = jnp.concatenate(
        [ap[None], bp[None], an[None], bn[None]], axis=0)[None]


def _tc_prep(node_reps, basis, att, mapping_func):
    blk = 1000
    grid = NN // blk
    return pl.pallas_call(
        _tc_prep_body,
        grid=(grid,),
        in_specs=[
            pl.BlockSpec((blk, D), lambda i: (i, 0)),
            pl.BlockSpec((2, D, D), lambda i: (0, 0, 0)),
            pl.BlockSpec((2, 2), lambda i: (0, 0)),
            pl.BlockSpec((1, 2 * D), lambda i: (0, 0)),
        ],
        out_specs=[
            pl.BlockSpec((blk, D), lambda i: (i, 0)),
            pl.BlockSpec((blk, D), lambda i: (i, 0)),
            pl.BlockSpec((1, 4, blk), lambda i: (i, 0, 0)),
        ],
        out_shape=[
            jax.ShapeDtypeStruct((NN, D), jnp.float32),
            jax.ShapeDtypeStruct((NN, D), jnp.float32),
            jax.ShapeDtypeStruct((NN // blk, 4, blk), jnp.float32),
        ],
    )(node_reps, basis, att, mapping_func)


def _sc_deg_body(srcs_hbm, out_hbm, idx_v, ones_v, zb_v, deg_sh, sem):
    c = lax.axis_index("c")
    s = lax.axis_index("s")
    w = c * NS + s
    for i in range(PER_TILE // L):
        zb_v[pl.ds(i * L, L)] = jnp.zeros((L,), jnp.float32)
    for i in range(BLK // L):
        ones_v[pl.ds(i * L, L)] = jnp.ones((L,), jnp.float32)
    pltpu.sync_copy(zb_v, deg_sh.at[pl.ds(s * PER_TILE, PER_TILE)])
    plsc.subcore_barrier()

    @pl.loop(w, NDROW // DSTG, step=NW)
    def _(t):
        pltpu.sync_copy(srcs_hbm.at[t], idx_v)
        for k in range(DSTG):
            pltpu.make_async_copy(
                ones_v, deg_sh.at[idx_v.at[k]], sem).start(add=True)
        for k in range(DSTG):
            pltpu.make_async_copy(
                ones_v, deg_sh.at[idx_v.at[k]], sem).wait()

    plsc.subcore_barrier()
    pltpu.sync_copy(deg_sh.at[pl.ds(s * PER_TILE, PER_TILE)],
                    out_hbm.at[c, pl.ds(s * PER_TILE, PER_TILE)])


def _sc_deg(srcs2d):
    mesh = plsc.VectorSubcoreMesh(core_axis_name="c", subcore_axis_name="s",
                                  num_cores=NC, num_subcores=NS)
    f = pl.kernel(
        _sc_deg_body,
        out_type=jax.ShapeDtypeStruct((NC, NPAD), jnp.float32),
        mesh=mesh,
        compiler_params=pltpu.CompilerParams(needs_layout_passes=False),
        scratch_types=[
            pltpu.VMEM((DSTG, BLK), jnp.int32),
            pltpu.VMEM((BLK,), jnp.float32),
            pltpu.VMEM((PER_TILE,), jnp.float32),
            pltpu.VMEM_SHARED((NPAD,), jnp.float32),
            pltpu.SemaphoreType.DMA,
        ],
    )
    return f(srcs2d)


def _tc_inv_body(deg_ref, inv_ref):
    inv_ref[...] = lax.rsqrt(deg_ref[0:1, :] + deg_ref[1:2, :])


def _tc_inv(degp):
    return pl.pallas_call(
        _tc_inv_body,
        out_shape=jax.ShapeDtypeStruct((1, NPAD), jnp.float32),
    )(degp)


def _sc_coef_body(adjp_hbm, adjn_hbm, scal_hbm, inv_hbm, pkp_hbm, pkn_hbm,
                  tab_a, tab_b, tab_inv, idx_v, pkv):
    c = lax.axis_index("c")
    s = lax.axis_index("s")
    w = c * NS + s
    pltpu.sync_copy(inv_hbm.at[0], tab_inv)

    def sign_pass(adj_hbm, pk_hbm, r0, r1, sign):
        pltpu.sync_copy(scal_hbm.at[r0], tab_a)
        pltpu.sync_copy(scal_hbm.at[r1], tab_b)

        @pl.loop(w, NSTG, step=NW)
        def _(t):
            base = t * (CSTG * BLK)
            pltpu.sync_copy(adj_hbm.at[:, pl.ds(base, CSTG * BLK)], idx_v)
            for k in range(CSTG):
                for i in range(BLK // L):
                    off = k * BLK + i * L
                    s16 = idx_v[0, pl.ds(off, L)]
                    d16 = idx_v[1, pl.ds(off, L)]
                    va = plsc.load_gather(tab_a, [s16])
                    vb = plsc.load_gather(tab_b, [d16])
                    x = va + vb
                    x = jnp.maximum(x, 0.2 * x)
                    sg = sign / (1.0 + jnp.exp(-x))
                    vi = plsc.load_gather(tab_inv, [d16])
                    sl = pl.ds(i * L, L)
                    pkv[0, sl] = s16
                    pkv[1, sl] = d16
                    pkv[2, sl] = plsc.bitcast(sg * vi, jnp.int32)
                pltpu.sync_copy(pkv, pk_hbm.at[t * CSTG + k])

    sign_pass(adjp_hbm, pkp_hbm, 0, 1, 1.0)
    sign_pass(adjn_hbm, pkn_hbm, 2, 3, -1.0)


def _sc_coef(adj_pos, adj_neg, scal, inv):
    mesh = plsc.VectorSubcoreMesh(core_axis_name="c", subcore_axis_name="s",
                                  num_cores=NC, num_subcores=NS)
    f = pl.kernel(
        _sc_coef_body,
        out_type=[
            jax.ShapeDtypeStruct((NBLK, 3, BLK), jnp.int32),
            jax.ShapeDtypeStruct((NBLK, 3, BLK), jnp.int32),
        ],
        mesh=mesh,
        compiler_params=pltpu.CompilerParams(needs_layout_passes=False),
        scratch_types=[
            pltpu.VMEM((NN,), jnp.float32),
            pltpu.VMEM((NN,), jnp.float32),
            pltpu.VMEM((NPAD,), jnp.float32),
            pltpu.VMEM((2, CSTG * BLK), jnp.int32),
            pltpu.VMEM((3, BLK), jnp.int32),
        ],
    )
    return f(adj_pos, adj_neg, scal, inv)


def _sc_main_body(pkp_hbm, pkn_hbm, hp_hbm, hn_hbm, out_hbm,
                  pk0, pk1, pk2, pk3, rows0, rows1, acc_sh,
                  sp0, sp1, sp2, sp3, sg0, sg1, ss0, ss1):
    c = lax.axis_index("c")
    s = lax.axis_index("s")
    w = c * NS + s
    pkb = (pk0, pk1, pk2, pk3)
    spk = (sp0, sp1, sp2, sp3)
    rwb = (rows0, rows1)
    sgt = (sg0, sg1)
    ssb = (ss0, ss1)

    @pl.loop(0, BLK)
    def _(i):
        for j in range(D // L):
            rows0[i, pl.ds(j * L, L)] = jnp.zeros((L,), jnp.float32)

    for k in range(PER_TILE // BLK):
        pltpu.sync_copy(rows0, acc_sh.at[pl.ds(s * PER_TILE + k * BLK, BLK)])
    plsc.subcore_barrier()

    def phase(pk_hbm, h_hbm):
        def pk_cp(b, i):
            return pltpu.make_async_copy(pk_hbm.at[b], pkb[i], spk[i])

        def g_cp2(pi, ri):
            return pltpu.make_async_copy(
                h_hbm.at[pkb[pi].at[1]], rwb[ri], sgt[ri])

        def sc_cp(pi, ri):
            return pltpu.make_async_copy(
                rwb[ri], acc_sh.at[pkb[pi].at[0]], ssb[ri])

        def proc(pi, ri):
            g_cp2(pi, ri).wait()
            two = jnp.broadcast_to(jnp.int32(2), (L,))

            @pl.loop(0, BLK, unroll=4)
            def _(e):
                e16 = jnp.broadcast_to(e, (L,)).astype(jnp.int32)
                cv = plsc.bitcast(
                    plsc.load_gather(pkb[pi], [two, e16]), jnp.float32)
                for j in range(D // L):
                    sl = pl.ds(j * L, L)
                    rwb[ri][e, sl] = rwb[ri][e, sl] * cv

            sc_cp(pi, ri).start(add=True)

        for q in range(4):
            pk_cp(w + q * NW, q).start()
        pk_cp(w, 0).wait()
        g_cp2(0, 0).start()

        @pl.loop(0, KMAX // 4)
        def _(q):
            b0 = w + (4 * q) * NW
            for u in range(4):
                bu = b0 + u * NW
                bn = bu + NW
                bp = bu - NW

                @pl.when(bn < NBLK)
                def _(bn=bn, bp=bp, u=u):
                    pk_cp(bn, (u + 1) % 4).wait()

                    @pl.when(bp >= 0)
                    def _(bp=bp, u=u):
                        sc_cp((u - 1) % 4, (u + 1) % 2).wait()

                    g_cp2((u + 1) % 4, (u + 1) % 2).start()

                @pl.when(bu < NBLK)
                def _(bu=bu, u=u):
                    proc(u % 4, u % 2)

                @pl.when((bp >= 0) & (bp + 4 * NW < NBLK))
                def _(bp=bp, u=u):
                    pk_cp(bp + 4 * NW, (u - 1) % 4).start()

        for v in range(KMAX):
            bv = w + v * NW

            @pl.when((bv < NBLK) & (bv + 2 * NW >= NBLK))
            def _(v=v):
                sc_cp(v % 4, v % 2).wait()

    phase(pkp_hbm, hp_hbm)
    phase(pkn_hbm, hn_hbm)
    plsc.subcore_barrier()
    pltpu.sync_copy(acc_sh.at[pl.ds(s * PER_TILE, PER_TILE)],
                    out_hbm.at[c, pl.ds(s * PER_TILE, PER_TILE)])


def _sc_main(pk_pos, pk_neg, h_pos, h_neg):
    mesh = plsc.VectorSubcoreMesh(core_axis_name="c", subcore_axis_name="s",
                                  num_cores=NC, num_subcores=NS)
    f = pl.kernel(
        _sc_main_body,
        out_type=jax.ShapeDtypeStruct((NC, NPAD, D), jnp.float32),
        mesh=mesh,
        compiler_params=pltpu.CompilerParams(needs_layout_passes=False),
        scratch_types=[
            pltpu.VMEM((3, BLK), jnp.int32),
            pltpu.VMEM((3, BLK), jnp.int32),
            pltpu.VMEM((3, BLK), jnp.int32),
            pltpu.VMEM((3, BLK), jnp.int32),
            pltpu.VMEM((BLK, D), jnp.float32),
            pltpu.VMEM((BLK, D), jnp.float32),
            pltpu.VMEM_SHARED((NPAD, D), jnp.float32),
            pltpu.SemaphoreType.DMA,
            pltpu.SemaphoreType.DMA,
            pltpu.SemaphoreType.DMA,
            pltpu.SemaphoreType.DMA,
            pltpu.SemaphoreType.DMA,
            pltpu.SemaphoreType.DMA,
            pltpu.SemaphoreType.DMA,
            pltpu.SemaphoreType.DMA,
        ],
    )
    return f(pk_pos, pk_neg, h_pos, h_neg)


def _tc_final_body(p_ref, deg_ref, bias_ref, out_ref):
    deg = deg_ref[0] + deg_ref[1]
    inv = jnp.where(deg > 0.0, lax.rsqrt(deg), 0.0)
    out_ref[...] = (p_ref[0] + p_ref[1]) * inv[:, None] + bias_ref[...]


def _tc_final(partials, degp, bias):
    blk = 2048
    grid = NPAD // blk
    return pl.pallas_call(
        _tc_final_body,
        grid=(grid,),
        in_specs=[
            pl.BlockSpec((NC, blk, D), lambda i: (0, i, 0)),
            pl.BlockSpec((NC, blk), lambda i: (0, i)),
            pl.BlockSpec((1, D), lambda i: (0, 0)),
        ],
        out_specs=pl.BlockSpec((blk, D), lambda i: (i, 0)),
        out_shape=jax.ShapeDtypeStruct((NPAD, D), jnp.float32),
    )(partials, degp, bias)


def kernel(node_reps, adj_pos, adj_neg, basis, att, mapping_func, bias):
    h_pos, h_neg, scal3 = _tc_prep(node_reps, basis, att, mapping_func)
    scal = scal3.transpose(1, 0, 2).reshape(4, NN)
    srcs2d = jnp.concatenate([adj_pos[0], adj_neg[0]]).reshape(
        NDROW // DSTG, DSTG, BLK)
    degp = _sc_deg(srcs2d)
    inv = _tc_inv(degp)
    pk_pos, pk_neg = _sc_coef(adj_pos, adj_neg, scal, inv)
    partials = _sc_main(pk_pos, pk_neg, h_pos, h_neg)
    return _tc_final(partials, degp, bias)[:NN]


# speedup vs baseline: 1.0569x; 1.0069x over previous
import jax
import jax.numpy as jnp
from jax import lax
from jax.experimental import pallas as pl
from jax.experimental.pallas import tpu as pltpu
from jax.experimental.pallas import tpu_sc as plsc

NN = 10000
NPAD = 10240
EE = 160000
D = 128
BLK = 128
NBLK = EE // BLK
NC = 2
NS = 16
NW = NC * NS
L = 16
PER_TILE = NPAD // NS
KMAX = 40
CSTG = 10
NSTG = NBLK // CSTG
DSTG = 10
NDROW = 2 * EE // BLK


def _tc_prep_body(node_ref, basis_ref, att_ref, mf_ref, hp_ref, hn_ref,
                  scal_ref):
    x = node_ref[...]
    att = att_ref[...]
    b0 = basis_ref[0]
    b1 = basis_ref[1]
    w0 = att[0:1, 0:1] * b0 + att[0:1, 1:2] * b1
    w1 = att[1:2, 0:1] * b0 + att[1:2, 1:2] * b1
    hp = jnp.dot(x, w0, preferred_element_type=jnp.float32)
    hn = jnp.dot(x, w1, preferred_element_type=jnp.float32)
    hp_ref[...] = hp
    hn_ref[...] = hn
    mf = mf_ref[...]
    mfa = mf[:, :D]
    mfb = mf[:, D:]
    ap = jnp.sum(hp * mfa, axis=1)
    bp = jnp.sum(hp * mfb, axis=1)
    an = jnp.sum(hn * mfa, axis=1)
    bn = jnp.sum(hn * mfb, axis=1)
    scal_ref[...] = jnp.concatenate(
        [ap[None], bp[None], an[None], bn[None]], axis=0)[None]


def _tc_prep(node_reps, basis, att, mapping_func):
    blk = 1000
    grid = NN // blk
    return pl.pallas_call(
        _tc_prep_body,
        grid=(grid,),
        in_specs=[
            pl.BlockSpec((blk, D), lambda i: (i, 0)),
            pl.BlockSpec((2, D, D), lambda i: (0, 0, 0)),
            pl.BlockSpec((2, 2), lambda i: (0, 0)),
            pl.BlockSpec((1, 2 * D), lambda i: (0, 0)),
        ],
        out_specs=[
            pl.BlockSpec((blk, D), lambda i: (i, 0)),
            pl.BlockSpec((blk, D), lambda i: (i, 0)),
            pl.BlockSpec((1, 4, blk), lambda i: (i, 0, 0)),
        ],
        out_shape=[
            jax.ShapeDtypeStruct((NN, D), jnp.float32),
            jax.ShapeDtypeStruct((NN, D), jnp.float32),
            jax.ShapeDtypeStruct((NN // blk, 4, blk), jnp.float32),
        ],
    )(node_reps, basis, att, mapping_func)


def _sc_deg_body(srcs_hbm, out_hbm, idx_v, ones_v, zb_v, deg_sh, sem):
    c = lax.axis_index("c")
    s = lax.axis_index("s")
    w = c * NS + s
    for i in range(PER_TILE // L):
        zb_v[pl.ds(i * L, L)] = jnp.zeros((L,), jnp.float32)
    for i in range(BLK // L):
        ones_v[pl.ds(i * L, L)] = jnp.ones((L,), jnp.float32)
    pltpu.sync_copy(zb_v, deg_sh.at[pl.ds(s * PER_TILE, PER_TILE)])
    plsc.subcore_barrier()

    @pl.loop(w, NDROW // DSTG, step=NW)
    def _(t):
        pltpu.sync_copy(srcs_hbm.at[t], idx_v)
        for k in range(DSTG):
            pltpu.make_async_copy(
                ones_v, deg_sh.at[idx_v.at[k]], sem).start(add=True)
        for k in range(DSTG):
            pltpu.make_async_copy(
                ones_v, deg_sh.at[idx_v.at[k]], sem).wait()

    plsc.subcore_barrier()
    pltpu.sync_copy(deg_sh.at[pl.ds(s * PER_TILE, PER_TILE)],
                    out_hbm.at[c, pl.ds(s * PER_TILE, PER_TILE)])


def _sc_deg(srcs2d):
    mesh = plsc.VectorSubcoreMesh(core_axis_name="c", subcore_axis_name="s",
                                  num_cores=NC, num_subcores=NS)
    f = pl.kernel(
        _sc_deg_body,
        out_type=jax.ShapeDtypeStruct((NC, NPAD), jnp.float32),
        mesh=mesh,
        compiler_params=pltpu.CompilerParams(needs_layout_passes=False),
        scratch_types=[
            pltpu.VMEM((DSTG, BLK), jnp.int32),
            pltpu.VMEM((BLK,), jnp.float32),
            pltpu.VMEM((PER_TILE,), jnp.float32),
            pltpu.VMEM_SHARED((NPAD,), jnp.float32),
            pltpu.SemaphoreType.DMA,
        ],
    )
    return f(srcs2d)


def _tc_inv_body(deg_ref, inv_ref):
    inv_ref[...] = lax.rsqrt(deg_ref[0:1, :] + deg_ref[1:2, :])


def _tc_inv(degp):
    return pl.pallas_call(
        _tc_inv_body,
        out_shape=jax.ShapeDtypeStruct((1, NPAD), jnp.float32),
    )(degp)


def _sc_coef_body(adjp_hbm, adjn_hbm, scal_hbm, inv_hbm, pkp_hbm, pkn_hbm,
                  tab_a, tab_b, tab_inv, idx_v, pkv):
    c = lax.axis_index("c")
    s = lax.axis_index("s")
    w = c * NS + s
    pltpu.sync_copy(inv_hbm.at[0], tab_inv)

    def sign_pass(adj_hbm, pk_hbm, r0, r1, sign):
        pltpu.sync_copy(scal_hbm.at[r0], tab_a)
        pltpu.sync_copy(scal_hbm.at[r1], tab_b)

        @pl.loop(w, NSTG, step=NW)
        def _(t):
            base = t * (CSTG * BLK)
            pltpu.sync_copy(adj_hbm.at[:, pl.ds(base, CSTG * BLK)], idx_v)
            for k in range(CSTG):
                for i in range(BLK // L):
                    off = k * BLK + i * L
                    s16 = idx_v[0, pl.ds(off, L)]
                    d16 = idx_v[1, pl.ds(off, L)]
                    va = plsc.load_gather(tab_a, [s16])
                    vb = plsc.load_gather(tab_b, [d16])
                    x = va + vb
                    x = jnp.maximum(x, 0.2 * x)
                    sg = sign / (1.0 + jnp.exp(-x))
                    vi = (plsc.load_gather(tab_inv, [s16]) *
                          plsc.load_gather(tab_inv, [d16]))
                    sl = pl.ds(i * L, L)
                    pkv[0, sl] = s16
                    pkv[1, sl] = d16
                    pkv[2, sl] = plsc.bitcast(sg * vi, jnp.int32)
                pltpu.sync_copy(pkv, pk_hbm.at[t * CSTG + k])

    sign_pass(adjp_hbm, pkp_hbm, 0, 1, 1.0)
    sign_pass(adjn_hbm, pkn_hbm, 2, 3, -1.0)


def _sc_coef(adj_pos, adj_neg, scal, inv):
    mesh = plsc.VectorSubcoreMesh(core_axis_name="c", subcore_axis_name="s",
                                  num_cores=NC, num_subcores=NS)
    f = pl.kernel(
        _sc_coef_body,
        out_type=[
            jax.ShapeDtypeStruct((NBLK, 3, BLK), jnp.int32),
            jax.ShapeDtypeStruct((NBLK, 3, BLK), jnp.int32),
        ],
        mesh=mesh,
        compiler_params=pltpu.CompilerParams(needs_layout_passes=False),
        scratch_types=[
            pltpu.VMEM((NN,), jnp.float32),
            pltpu.VMEM((NN,), jnp.float32),
            pltpu.VMEM((NPAD,), jnp.float32),
            pltpu.VMEM((2, CSTG * BLK), jnp.int32),
            pltpu.VMEM((3, BLK), jnp.int32),
        ],
    )
    return f(adj_pos, adj_neg, scal, inv)


def _sc_main_body(pkp_hbm, pkn_hbm, hp_hbm, hn_hbm, out_hbm,
                  pk0, pk1, pk2, pk3, rows0, rows1, acc_sh,
                  sp0, sp1, sp2, sp3, sg0, sg1, ss0, ss1):
    c = lax.axis_index("c")
    s = lax.axis_index("s")
    w = c * NS + s
    pkb = (pk0, pk1, pk2, pk3)
    spk = (sp0, sp1, sp2, sp3)
    rwb = (rows0, rows1)
    sgt = (sg0, sg1)
    ssb = (ss0, ss1)

    @pl.loop(0, BLK)
    def _(i):
        for j in range(D // L):
            rows0[i, pl.ds(j * L, L)] = jnp.zeros((L,), jnp.float32)

    for k in range(PER_TILE // BLK):
        pltpu.sync_copy(rows0, acc_sh.at[pl.ds(s * PER_TILE + k * BLK, BLK)])
    plsc.subcore_barrier()

    def phase(pk_hbm, h_hbm):
        def pk_cp(b, i):
            return pltpu.make_async_copy(pk_hbm.at[b], pkb[i], spk[i])

        def g_cp2(pi, ri):
            return pltpu.make_async_copy(
                h_hbm.at[pkb[pi].at[1]], rwb[ri], sgt[ri])

        def sc_cp(pi, ri):
            return pltpu.make_async_copy(
                rwb[ri], acc_sh.at[pkb[pi].at[0]], ssb[ri])

        def proc(pi, ri):
            g_cp2(pi, ri).wait()
            two = jnp.broadcast_to(jnp.int32(2), (L,))

            @pl.loop(0, BLK, unroll=4)
            def _(e):
                e16 = jnp.broadcast_to(e, (L,)).astype(jnp.int32)
                cv = plsc.bitcast(
                    plsc.load_gather(pkb[pi], [two, e16]), jnp.float32)
                for j in range(D // L):
                    sl = pl.ds(j * L, L)
                    rwb[ri][e, sl] = rwb[ri][e, sl] * cv

            sc_cp(pi, ri).start(add=True)

        for q in range(4):
            pk_cp(w + q * NW, q).start()
        pk_cp(w, 0).wait()
        g_cp2(0, 0).start()

        @pl.loop(0, KMAX // 4)
        def _(q):
            b0 = w + (4 * q) * NW
            for u in range(4):
                bu = b0 + u * NW
                bn = bu + NW
                bp = bu - NW

                @pl.when(bn < NBLK)
                def _(bn=bn, bp=bp, u=u):
                    pk_cp(bn, (u + 1) % 4).wait()

                    @pl.when(bp >= 0)
                    def _(bp=bp, u=u):
                        sc_cp((u - 1) % 4, (u + 1) % 2).wait()

                    g_cp2((u + 1) % 4, (u + 1) % 2).start()

                @pl.when(bu < NBLK)
                def _(bu=bu, u=u):
                    proc(u % 4, u % 2)

                @pl.when((bp >= 0) & (bp + 4 * NW < NBLK))
                def _(bp=bp, u=u):
                    pk_cp(bp + 4 * NW, (u - 1) % 4).start()

        for v in range(KMAX):
            bv = w + v * NW

            @pl.when((bv < NBLK) & (bv + 2 * NW >= NBLK))
            def _(v=v):
                sc_cp(v % 4, v % 2).wait()

    phase(pkp_hbm, hp_hbm)
    phase(pkn_hbm, hn_hbm)
    plsc.subcore_barrier()
    pltpu.sync_copy(acc_sh.at[pl.ds(s * PER_TILE, PER_TILE)],
                    out_hbm.at[c, pl.ds(s * PER_TILE, PER_TILE)])


def _sc_main(pk_pos, pk_neg, h_pos, h_neg):
    mesh = plsc.VectorSubcoreMesh(core_axis_name="c", subcore_axis_name="s",
                                  num_cores=NC, num_subcores=NS)
    f = pl.kernel(
        _sc_main_body,
        out_type=jax.ShapeDtypeStruct((NC, NPAD, D), jnp.float32),
        mesh=mesh,
        compiler_params=pltpu.CompilerParams(needs_layout_passes=False),
        scratch_types=[
            pltpu.VMEM((3, BLK), jnp.int32),
            pltpu.VMEM((3, BLK), jnp.int32),
            pltpu.VMEM((3, BLK), jnp.int32),
            pltpu.VMEM((3, BLK), jnp.int32),
            pltpu.VMEM((BLK, D), jnp.float32),
            pltpu.VMEM((BLK, D), jnp.float32),
            pltpu.VMEM_SHARED((NPAD, D), jnp.float32),
            pltpu.SemaphoreType.DMA,
            pltpu.SemaphoreType.DMA,
            pltpu.SemaphoreType.DMA,
            pltpu.SemaphoreType.DMA,
            pltpu.SemaphoreType.DMA,
            pltpu.SemaphoreType.DMA,
            pltpu.SemaphoreType.DMA,
            pltpu.SemaphoreType.DMA,
        ],
    )
    return f(pk_pos, pk_neg, h_pos, h_neg)


def _tc_final_body(p_ref, bias_ref, out_ref):
    out_ref[...] = p_ref[0] + p_ref[1] + bias_ref[...]


def _tc_final(partials, bias):
    blk = 2000
    grid = NN // blk
    return pl.pallas_call(
        _tc_final_body,
        grid=(grid,),
        in_specs=[
            pl.BlockSpec((NC, blk, D), lambda i: (0, i, 0)),
            pl.BlockSpec((1, D), lambda i: (0, 0)),
        ],
        out_specs=pl.BlockSpec((blk, D), lambda i: (i, 0)),
        out_shape=jax.ShapeDtypeStruct((NN, D), jnp.float32),
    )(partials, bias)


def kernel(node_reps, adj_pos, adj_neg, basis, att, mapping_func, bias):
    h_pos, h_neg, scal3 = _tc_prep(node_reps, basis, att, mapping_func)
    scal = scal3.transpose(1, 0, 2).reshape(4, NN)
    srcs2d = jnp.concatenate([adj_pos[0], adj_neg[0]]).reshape(
        NDROW // DSTG, DSTG, BLK)
    degp = _sc_deg(srcs2d)
    inv = _tc_inv(degp)
    pk_pos, pk_neg = _sc_coef(adj_pos, adj_neg, scal, inv)
    partials = _sc_main(pk_pos, pk_neg, h_pos, h_neg)
    return _tc_final(partials, bias)


# speedup vs baseline: 1.0805x; 1.0223x over previous
import jax
import jax.numpy as jnp
from jax import lax
from jax.experimental import pallas as pl
from jax.experimental.pallas import tpu as pltpu
from jax.experimental.pallas import tpu_sc as plsc

NN = 10000
NPAD = 10240
EE = 160000
D = 128
BLK = 128
NBLK = EE // BLK
NC = 2
NS = 16
NW = NC * NS
L = 16
PER_TILE = NPAD // NS
KMAX = 40
CSTG = 10
NSTG = NBLK // CSTG
DSTG = 10
NDROW = 2 * EE // BLK


def _tc_prep_body(node_ref, basis_ref, att_ref, mf_ref, hp_ref, hn_ref,
                  scal_ref):
    x = node_ref[...]
    att = att_ref[...]
    b0 = basis_ref[0]
    b1 = basis_ref[1]
    w0 = att[0:1, 0:1] * b0 + att[0:1, 1:2] * b1
    w1 = att[1:2, 0:1] * b0 + att[1:2, 1:2] * b1
    hp = jnp.dot(x, w0, preferred_element_type=jnp.float32)
    hn = jnp.dot(x, w1, preferred_element_type=jnp.float32)
    hp_ref[...] = hp
    hn_ref[...] = hn
    mf = mf_ref[...]
    mfa = mf[:, :D]
    mfb = mf[:, D:]
    ap = jnp.sum(hp * mfa, axis=1)
    bp = jnp.sum(hp * mfb, axis=1)
    an = jnp.sum(hn * mfa, axis=1)
    bn = jnp.sum(hn * mfb, axis=1)
    scal_ref[...] = jnp.concatenate(
        [ap[None], bp[None], an[None], bn[None]], axis=0)[None]


def _tc_prep(node_reps, basis, att, mapping_func):
    blk = 1000
    grid = NN // blk
    return pl.pallas_call(
        _tc_prep_body,
        grid=(grid,),
        in_specs=[
            pl.BlockSpec((blk, D), lambda i: (i, 0)),
            pl.BlockSpec((2, D, D), lambda i: (0, 0, 0)),
            pl.BlockSpec((2, 2), lambda i: (0, 0)),
            pl.BlockSpec((1, 2 * D), lambda i: (0, 0)),
        ],
        out_specs=[
            pl.BlockSpec((blk, D), lambda i: (i, 0)),
            pl.BlockSpec((blk, D), lambda i: (i, 0)),
            pl.BlockSpec((1, 4, blk), lambda i: (i, 0, 0)),
        ],
        out_shape=[
            jax.ShapeDtypeStruct((NN, D), jnp.float32),
            jax.ShapeDtypeStruct((NN, D), jnp.float32),
            jax.ShapeDtypeStruct((NN // blk, 4, blk), jnp.float32),
        ],
    )(node_reps, basis, att, mapping_func)


def _sc_deg_body(srcs_hbm, out_hbm, idx_v, ones_v, zb_v, deg_sh, sem):
    c = lax.axis_index("c")
    s = lax.axis_index("s")
    w = c * NS + s
    for i in range(PER_TILE // L):
        zb_v[pl.ds(i * L, L)] = jnp.zeros((L,), jnp.float32)
    for i in range(BLK // L):
        ones_v[pl.ds(i * L, L)] = jnp.ones((L,), jnp.float32)
    pltpu.sync_copy(zb_v, deg_sh.at[pl.ds(s * PER_TILE, PER_TILE)])
    plsc.subcore_barrier()

    @pl.loop(w, NDROW // DSTG, step=NW)
    def _(t):
        pltpu.sync_copy(srcs_hbm.at[t], idx_v)
        for k in range(DSTG):
            pltpu.make_async_copy(
                ones_v, deg_sh.at[idx_v.at[k]], sem).start(add=True)
        for k in range(DSTG):
            pltpu.make_async_copy(
                ones_v, deg_sh.at[idx_v.at[k]], sem).wait()

    plsc.subcore_barrier()
    pltpu.sync_copy(deg_sh.at[pl.ds(s * PER_TILE, PER_TILE)],
                    out_hbm.at[c, pl.ds(s * PER_TILE, PER_TILE)])


def _sc_deg(srcs2d):
    mesh = plsc.VectorSubcoreMesh(core_axis_name="c", subcore_axis_name="s",
                                  num_cores=NC, num_subcores=NS)
    f = pl.kernel(
        _sc_deg_body,
        out_type=jax.ShapeDtypeStruct((NC, NPAD), jnp.float32),
        mesh=mesh,
        compiler_params=pltpu.CompilerParams(needs_layout_passes=False),
        scratch_types=[
            pltpu.VMEM((DSTG, BLK), jnp.int32),
            pltpu.VMEM((BLK,), jnp.float32),
            pltpu.VMEM((PER_TILE,), jnp.float32),
            pltpu.VMEM_SHARED((NPAD,), jnp.float32),
            pltpu.SemaphoreType.DMA,
        ],
    )
    return f(srcs2d)


def _tc_inv_body(deg_ref, inv_ref):
    inv_ref[...] = lax.rsqrt(deg_ref[0:1, :] + deg_ref[1:2, :])


def _tc_inv(degp):
    return pl.pallas_call(
        _tc_inv_body,
        out_shape=jax.ShapeDtypeStruct((1, NPAD), jnp.float32),
    )(degp)


def _sc_coef_body(adjp_hbm, adjn_hbm, scal_hbm, inv_hbm, pkp_hbm, pkn_hbm,
                  tab_a, tab_b, tab_inv, idx2, pkv0, pkv1,
                  si0, si1, so0, so1):
    c = lax.axis_index("c")
    s = lax.axis_index("s")
    w = c * NS + s
    pkvb = (pkv0, pkv1)
    sib = (si0, si1)
    sob = (so0, so1)
    pltpu.sync_copy(inv_hbm.at[0], tab_inv)
    KCOEF = (NSTG + NW - 1) // NW

    def sign_pass(adj_hbm, pk_hbm, r0, r1, sign):
        def idx_cp(t, q):
            base = t * (CSTG * BLK)
            return pltpu.make_async_copy(
                adj_hbm.at[:, pl.ds(base, CSTG * BLK)],
                idx2.at[pl.ds(2 * q, 2)], sib[q])

        idx_cp(w, 0).start()
        pltpu.sync_copy(scal_hbm.at[r0], tab_a)
        pltpu.sync_copy(scal_hbm.at[r1], tab_b)

        @pl.loop(0, KCOEF)
        def _(q):
            t = w + q * NW
            nt = t + NW
            par = q % 2

            @pl.when((nt < NSTG) & (par == 0))
            def _(nt=nt):
                idx_cp(nt, 1).start()

            @pl.when((nt < NSTG) & (par == 1))
            def _(nt=nt):
                idx_cp(nt, 0).start()

            @pl.when(t < NSTG)
            def _(t=t, par=par):
                @pl.when(par == 0)
                def _(t=t):
                    idx_cp(t, 0).wait()

                @pl.when(par == 1)
                def _(t=t):
                    idx_cp(t, 1).wait()

                row = 2 * par
                for k in range(CSTG):
                    pkv = pkvb[k % 2]
                    out_cp = pltpu.make_async_copy(
                        pkv, pk_hbm.at[t * CSTG + k], sob[k % 2])
                    if k >= 2:
                        pltpu.make_async_copy(
                            pkvb[k % 2], pk_hbm.at[t * CSTG + k - 2],
                            sob[k % 2]).wait()
                    for i in range(BLK // L):
                        off = k * BLK + i * L
                        s16 = idx2[row, pl.ds(off, L)]
                        d16 = idx2[row + 1, pl.ds(off, L)]
                        va = plsc.load_gather(tab_a, [s16])
                        vb = plsc.load_gather(tab_b, [d16])
                        x = va + vb
                        x = jnp.maximum(x, 0.2 * x)
                        sg = sign / (1.0 + jnp.exp(-x))
                        vi = (plsc.load_gather(tab_inv, [s16]) *
                              plsc.load_gather(tab_inv, [d16]))
                        sl = pl.ds(i * L, L)
                        pkv[0, sl] = s16
                        pkv[1, sl] = d16
                        pkv[2, sl] = plsc.bitcast(sg * vi, jnp.int32)
                    out_cp.start()
                for k in (CSTG - 2, CSTG - 1):
                    pltpu.make_async_copy(
                        pkvb[k % 2], pk_hbm.at[t * CSTG + k],
                        sob[k % 2]).wait()

    sign_pass(adjp_hbm, pkp_hbm, 0, 1, 1.0)
    sign_pass(adjn_hbm, pkn_hbm, 2, 3, -1.0)


def _sc_coef(adj_pos, adj_neg, scal, inv):
    mesh = plsc.VectorSubcoreMesh(core_axis_name="c", subcore_axis_name="s",
                                  num_cores=NC, num_subcores=NS)
    f = pl.kernel(
        _sc_coef_body,
        out_type=[
            jax.ShapeDtypeStruct((NBLK, 3, BLK), jnp.int32),
            jax.ShapeDtypeStruct((NBLK, 3, BLK), jnp.int32),
        ],
        mesh=mesh,
        compiler_params=pltpu.CompilerParams(needs_layout_passes=False),
        scratch_types=[
            pltpu.VMEM((NN,), jnp.float32),
            pltpu.VMEM((NN,), jnp.float32),
            pltpu.VMEM((NPAD,), jnp.float32),
            pltpu.VMEM((4, CSTG * BLK), jnp.int32),
            pltpu.VMEM((3, BLK), jnp.int32),
            pltpu.VMEM((3, BLK), jnp.int32),
            pltpu.SemaphoreType.DMA,
            pltpu.SemaphoreType.DMA,
            pltpu.SemaphoreType.DMA,
            pltpu.SemaphoreType.DMA,
        ],
    )
    return f(adj_pos, adj_neg, scal, inv)


def _sc_main_body(pkp_hbm, pkn_hbm, hp_hbm, hn_hbm, out_hbm,
                  pk0, pk1, pk2, pk3, rows0, rows1, acc_sh,
                  sp0, sp1, sp2, sp3, sg0, sg1, ss0, ss1):
    c = lax.axis_index("c")
    s = lax.axis_index("s")
    w = c * NS + s
    pkb = (pk0, pk1, pk2, pk3)
    spk = (sp0, sp1, sp2, sp3)
    rwb = (rows0, rows1)
    sgt = (sg0, sg1)
    ssb = (ss0, ss1)

    @pl.loop(0, BLK)
    def _(i):
        for j in range(D // L):
            rows0[i, pl.ds(j * L, L)] = jnp.zeros((L,), jnp.float32)

    for k in range(PER_TILE // BLK):
        pltpu.sync_copy(rows0, acc_sh.at[pl.ds(s * PER_TILE + k * BLK, BLK)])
    plsc.subcore_barrier()

    def phase(pk_hbm, h_hbm):
        def pk_cp(b, i):
            return pltpu.make_async_copy(pk_hbm.at[b], pkb[i], spk[i])

        def g_cp2(pi, ri):
            return pltpu.make_async_copy(
                h_hbm.at[pkb[pi].at[1]], rwb[ri], sgt[ri])

        def sc_cp(pi, ri):
            return pltpu.make_async_copy(
                rwb[ri], acc_sh.at[pkb[pi].at[0]], ssb[ri])

        def proc(pi, ri):
            g_cp2(pi, ri).wait()
            two = jnp.broadcast_to(jnp.int32(2), (L,))

            @pl.loop(0, BLK, unroll=4)
            def _(e):
                e16 = jnp.broadcast_to(e, (L,)).astype(jnp.int32)
                cv = plsc.bitcast(
                    plsc.load_gather(pkb[pi], [two, e16]), jnp.float32)
                for j in range(D // L):
                    sl = pl.ds(j * L, L)
                    rwb[ri][e, sl] = rwb[ri][e, sl] * cv

            sc_cp(pi, ri).start(add=True)

        for q in range(4):
            pk_cp(w + q * NW, q).start()
        pk_cp(w, 0).wait()
        g_cp2(0, 0).start()

        @pl.loop(0, KMAX // 4)
        def _(q):
            b0 = w + (4 * q) * NW
            for u in range(4):
                bu = b0 + u * NW
                bn = bu + NW
                bp = bu - NW

                @pl.when(bn < NBLK)
                def _(bn=bn, bp=bp, u=u):
                    pk_cp(bn, (u + 1) % 4).wait()

                    @pl.when(bp >= 0)
                    def _(bp=bp, u=u):
                        sc_cp((u - 1) % 4, (u + 1) % 2).wait()

                    g_cp2((u + 1) % 4, (u + 1) % 2).start()

                @pl.when(bu < NBLK)
                def _(bu=bu, u=u):
                    proc(u % 4, u % 2)

                @pl.when((bp >= 0) & (bp + 4 * NW < NBLK))
                def _(bp=bp, u=u):
                    pk_cp(bp + 4 * NW, (u - 1) % 4).start()

        for v in range(KMAX):
            bv = w + v * NW

            @pl.when((bv < NBLK) & (bv + 2 * NW >= NBLK))
            def _(v=v):
                sc_cp(v % 4, v % 2).wait()

    phase(pkp_hbm, hp_hbm)
    phase(pkn_hbm, hn_hbm)
    plsc.subcore_barrier()
    pltpu.sync_copy(acc_sh.at[pl.ds(s * PER_TILE, PER_TILE)],
                    out_hbm.at[c, pl.ds(s * PER_TILE, PER_TILE)])


def _sc_main(pk_pos, pk_neg, h_pos, h_neg):
    mesh = plsc.VectorSubcoreMesh(core_axis_name="c", subcore_axis_name="s",
                                  num_cores=NC, num_subcores=NS)
    f = pl.kernel(
        _sc_main_body,
        out_type=jax.ShapeDtypeStruct((NC, NPAD, D), jnp.float32),
        mesh=mesh,
        compiler_params=pltpu.CompilerParams(needs_layout_passes=False),
        scratch_types=[
            pltpu.VMEM((3, BLK), jnp.int32),
            pltpu.VMEM((3, BLK), jnp.int32),
            pltpu.VMEM((3, BLK), jnp.int32),
            pltpu.VMEM((3, BLK), jnp.int32),
            pltpu.VMEM((BLK, D), jnp.float32),
            pltpu.VMEM((BLK, D), jnp.float32),
            pltpu.VMEM_SHARED((NPAD, D), jnp.float32),
            pltpu.SemaphoreType.DMA,
            pltpu.SemaphoreType.DMA,
            pltpu.SemaphoreType.DMA,
            pltpu.SemaphoreType.DMA,
            pltpu.SemaphoreType.DMA,
            pltpu.SemaphoreType.DMA,
            pltpu.SemaphoreType.DMA,
            pltpu.SemaphoreType.DMA,
        ],
    )
    return f(pk_pos, pk_neg, h_pos, h_neg)


def _tc_final_body(p_ref, bias_ref, out_ref):
    out_ref[...] = p_ref[0] + p_ref[1] + bias_ref[...]


def _tc_final(partials, bias):
    blk = 2000
    grid = NN // blk
    return pl.pallas_call(
        _tc_final_body,
        grid=(grid,),
        in_specs=[
            pl.BlockSpec((NC, blk, D), lambda i: (0, i, 0)),
            pl.BlockSpec((1, D), lambda i: (0, 0)),
        ],
        out_specs=pl.BlockSpec((blk, D), lambda i: (i, 0)),
        out_shape=jax.ShapeDtypeStruct((NN, D), jnp.float32),
    )(partials, bias)


def kernel(node_reps, adj_pos, adj_neg, basis, att, mapping_func, bias):
    h_pos, h_neg, scal3 = _tc_prep(node_reps, basis, att, mapping_func)
    scal = scal3.transpose(1, 0, 2).reshape(4, NN)
    srcs2d = jnp.concatenate([adj_pos[0], adj_neg[0]]).reshape(
        NDROW // DSTG, DSTG, BLK)
    degp = _sc_deg(srcs2d)
    inv = _tc_inv(degp)
    pk_pos, pk_neg = _sc_coef(adj_pos, adj_neg, scal, inv)
    partials = _sc_main(pk_pos, pk_neg, h_pos, h_neg)
    return _tc_final(partials, bias)
